# Initial kernel scaffold; baseline (speedup 1.0000x reference)
import jax
import jax.numpy as jnp
from jax.experimental import pallas as pl


def kernel(table, seq_len):
    raise NotImplementedError("write your pallas kernel here")


# speedup vs baseline: 6.3509x; 6.3509x over previous
import functools

import jax
import jax.numpy as jnp
from jax import lax
from jax.experimental import pallas as pl
from jax.experimental.pallas import tpu as pltpu
from jax.experimental.pallas import tpu_sc as plsc

MAX_REL = 128
VOCAB = 2 * MAX_REL + 1
HEAD_DIM = 64
SEQ = 2048
NUM_CORES = 2
NUM_SUBCORES = 16
NUM_WORKERS = NUM_CORES * NUM_SUBCORES
I_PER_W = SEQ // NUM_WORKERS
J_HALF = SEQ // 2
STRIP = J_HALF + I_PER_W - 1
STRIP_PAD = 1088
LANES = 16
CHUNKS = HEAD_DIM // LANES


def _rpe_body(table_hbm, out_hbm, table_v, strip_v):
    wid = lax.axis_index("s") * NUM_CORES + lax.axis_index("c")
    iw = wid * I_PER_W

    pltpu.sync_copy(table_hbm, table_v)

    for half in range(2):
        j0 = half * J_HALF
        base = (I_PER_W - 1) + iw - j0

        def build_row(r, _):
            idx = jnp.clip(base - r, -MAX_REL, MAX_REL) + MAX_REL
            for ch in range(CHUNKS):
                strip_v[pl.ds(r * HEAD_DIM + ch * LANES, LANES)] = (
                    table_v[pl.ds(idx * HEAD_DIM + ch * LANES, LANES)])
            return 0

        lax.fori_loop(0, STRIP_PAD, build_row, 0)

        def copy_row(rr, _):
            off = (I_PER_W - 1) - rr
            dst = (iw + rr) * (SEQ * HEAD_DIM) + j0 * HEAD_DIM
            pltpu.sync_copy(
                strip_v.at[pl.ds(off * HEAD_DIM, J_HALF * HEAD_DIM)],
                out_hbm.at[pl.ds(dst, J_HALF * HEAD_DIM)])
            return 0

        lax.fori_loop(0, I_PER_W, copy_row, 0)


_rpe = functools.partial(
    pl.kernel,
    out_type=jax.ShapeDtypeStruct((SEQ * SEQ * HEAD_DIM,), jnp.float32),
    mesh=plsc.VectorSubcoreMesh(core_axis_name="c", subcore_axis_name="s"),
    scratch_types=[
        pltpu.VMEM((VOCAB * HEAD_DIM,), jnp.float32),
        pltpu.VMEM((STRIP_PAD * HEAD_DIM,), jnp.float32),
    ],
)(_rpe_body)


def kernel(table, seq_len):
    del seq_len
    flat = _rpe(table.reshape(VOCAB * HEAD_DIM))
    return flat.reshape(SEQ, SEQ, HEAD_DIM)
